# baseline (speedup 1.0000x reference)
import functools

import jax
import jax.numpy as jnp
from jax import lax
from jax.experimental import pallas as pl
from jax.experimental.pallas import tpu as pltpu
from jax.experimental.pallas import tpu_sc as plsc

NA = 10000
NP = 320000
F = 128
PF = 16

NC = 2
NS = 16
NW = NC * NS
PER_TILE = NP // NW
CH = 80
NCHUNK = PER_TILE // CH

OCH = 80
N_OCH = NA // OCH


def _atoms_body(af, w1, w2, bap, waa, baa, t1, aa):
    x = af[...]
    u = jnp.dot(x, w1[...], preferred_element_type=jnp.float32)
    v = jnp.dot(x, w2[...], preferred_element_type=jnp.float32) + bap[...]
    ub = lax.bitcast_convert_type(u.astype(jnp.bfloat16),
                                  jnp.uint16).astype(jnp.uint32)
    vb = lax.bitcast_convert_type(v.astype(jnp.bfloat16),
                                  jnp.uint16).astype(jnp.uint32)
    t1[...] = (ub | (vb << 16)).astype(jnp.int32)
    aa[...] = jnp.maximum(
        jnp.dot(x, waa[...], preferred_element_type=jnp.float32) + baa[...], 0.0)


def _atoms_stage(atom_features, w1, w2, bap, waa, baa):
    blk = 2000
    return pl.pallas_call(
        _atoms_body,
        grid=(NA // blk,),
        in_specs=[
            pl.BlockSpec((blk, F), lambda i: (i, 0)),
            pl.BlockSpec((F, F), lambda i: (0, 0)),
            pl.BlockSpec((F, F), lambda i: (0, 0)),
            pl.BlockSpec((1, F), lambda i: (0, 0)),
            pl.BlockSpec((F, F), lambda i: (0, 0)),
            pl.BlockSpec((1, F), lambda i: (0, 0)),
        ],
        out_specs=[
            pl.BlockSpec((blk, F), lambda i: (i, 0)),
            pl.BlockSpec((blk, F), lambda i: (i, 0)),
        ],
        out_shape=[
            jax.ShapeDtypeStruct((NA, F), jnp.int32),
            jax.ShapeDtypeStruct((NA, F), jnp.float32),
        ],
    )(atom_features, w1, w2, bap, waa, baa)


_DN_T = (((0,), (0,)), ((), ()))


def _pa_body(pft, wpa, bpa, pa):
    pa[...] = jnp.maximum(
        lax.dot_general(pft[...], wpa[...], _DN_T,
                        preferred_element_type=jnp.float32)
        + bpa[...], 0.0)


def _pa_stage(pair_features_t, wpa, bpa):
    blk = 2560
    return pl.pallas_call(
        _pa_body,
        grid=(NP // blk,),
        in_specs=[
            pl.BlockSpec((PF, blk), lambda i: (0, i)),
            pl.BlockSpec((PF, F), lambda i: (0, 0)),
            pl.BlockSpec((1, F), lambda i: (0, 0)),
        ],
        out_specs=pl.BlockSpec((blk, F), lambda i: (i, 0)),
        out_shape=jax.ShapeDtypeStruct((NP, F), jnp.float32),
    )(pair_features_t, wpa, bpa)


def _sc_gather_body(uv1_hbm, ij_hbm, g_hbm,
                    iiall_v, jjall_v,
                    wi0, wj0, out0, wi1, wj1, out1,
                    gsem0, gsem1, osem0, osem1):
    cid = lax.axis_index("c")
    sid = lax.axis_index("s")
    wid = sid * NC + cid
    base = wid * PER_TILE

    pltpu.sync_copy(ij_hbm.at[pl.ds(base, PER_TILE)], iiall_v)
    pltpu.sync_copy(ij_hbm.at[pl.ds(NP + base, PER_TILE)], jjall_v)

    bufs = ((wi0, wj0, out0, gsem0, osem0), (wi1, wj1, out1, gsem1, osem1))

    def issue(k, b):
        wi, wj, _, gsem, _ = b
        c0 = k * CH
        pltpu.async_copy(uv1_hbm.at[iiall_v.at[pl.ds(c0, CH)]], wi, gsem)
        pltpu.async_copy(uv1_hbm.at[jjall_v.at[pl.ds(c0, CH)]], wj, gsem)

    def compute(k, b):
        wi, wj, out, gsem, osem = b
        c0 = k * CH
        off = base + c0
        pltpu.make_async_copy(uv1_hbm.at[iiall_v.at[pl.ds(c0, CH)]], wi,
                              gsem).wait()
        pltpu.make_async_copy(uv1_hbm.at[jjall_v.at[pl.ds(c0, CH)]], wj,
                              gsem).wait()

        @pl.when(k >= 2)
        def _():
            pltpu.make_async_copy(out, g_hbm.at[pl.ds(off, CH)], osem).wait()

        @plsc.parallel_loop(0, CH, 1, unroll=4)
        def pair(p):
            for h in range(F // 16):
                s = h * 16
                a = wi[p, pl.ds(s, 16)]
                bv = wj[p, pl.ds(s, 16)]
                ui = lax.bitcast_convert_type(a << 16, jnp.float32)
                vpi = lax.bitcast_convert_type(a & jnp.int32(-65536),
                                               jnp.float32)
                uj = lax.bitcast_convert_type(bv << 16, jnp.float32)
                vpj = lax.bitcast_convert_type(bv & jnp.int32(-65536),
                                               jnp.float32)
                out[p, pl.ds(s, 16)] = (jnp.maximum(ui + vpj, 0.0)
                                        + jnp.maximum(uj + vpi, 0.0))
        pltpu.async_copy(out, g_hbm.at[pl.ds(off, CH)], osem)

    issue(0, bufs[0])

    def body(m, carry):
        k0 = 2 * m
        issue(k0 + 1, bufs[1])
        compute(k0, bufs[0])
        issue(k0 + 2, bufs[0])
        compute(k0 + 1, bufs[1])
        return carry

    lax.fori_loop(0, (NCHUNK - 1) // 2, body, 0, unroll=False)
    compute(NCHUNK - 1, bufs[0])

    pltpu.make_async_copy(out0, g_hbm.at[pl.ds(base, CH)], osem0).wait()
    pltpu.make_async_copy(out1, g_hbm.at[pl.ds(base, CH)], osem1).wait()


_sc_gather = functools.partial(
    pl.kernel,
    out_type=jax.ShapeDtypeStruct((NP, F), jnp.float32),
    mesh=plsc.VectorSubcoreMesh(core_axis_name="c", subcore_axis_name="s"),
    scratch_types=[
        pltpu.VMEM((PER_TILE,), jnp.int32),
        pltpu.VMEM((PER_TILE,), jnp.int32),
        pltpu.VMEM((CH, F), jnp.int32),
        pltpu.VMEM((CH, F), jnp.int32),
        pltpu.VMEM((CH, F), jnp.float32),
        pltpu.VMEM((CH, F), jnp.int32),
        pltpu.VMEM((CH, F), jnp.int32),
        pltpu.VMEM((CH, F), jnp.float32),
        pltpu.SemaphoreType.DMA,
        pltpu.SemaphoreType.DMA,
        pltpu.SemaphoreType.DMA,
        pltpu.SemaphoreType.DMA,
    ],
)(_sc_gather_body)


def _sc_segsum_body(pa_hbm, split_hbm, out_hbm, acc_sh,
                    rows0_v, seg0_v, rows1_v, seg1_v, zbuf_v, lsem0, lsem1):
    cid = lax.axis_index("c")
    sid = lax.axis_index("s")
    wid = sid * NC + cid

    def zrow(p, c):
        for h in range(F // 16):
            zbuf_v[p, pl.ds(h * 16, 16)] = jnp.zeros((16,), jnp.float32)
        return c

    lax.fori_loop(0, OCH, zrow, 0, unroll=False)
    for z in range((N_OCH + NS - 1) // NS):
        cidx = sid + NS * z
        @pl.when(cidx < N_OCH)
        def _():
            pltpu.sync_copy(zbuf_v, acc_sh.at[pl.ds(cidx * OCH, OCH)])
    plsc.subcore_barrier()

    base = wid * PER_TILE
    bufs = ((rows0_v, seg0_v, lsem0), (rows1_v, seg1_v, lsem1))

    def issue(k, b):
        rows, seg, lsem = b
        off = base + k * CH
        pltpu.async_copy(pa_hbm.at[pl.ds(off, CH)], rows, lsem)
        pltpu.async_copy(split_hbm.at[pl.ds(off, CH)], seg, lsem)

    def scatter(k, b):
        rows, seg, lsem = b
        off = base + k * CH
        pltpu.make_async_copy(pa_hbm.at[pl.ds(off, CH)], rows, lsem).wait()
        pltpu.make_async_copy(split_hbm.at[pl.ds(off, CH)], seg, lsem).wait()
        pltpu.sync_copy(rows, acc_sh.at[seg], add=True)

    issue(0, bufs[0])

    def chunk2(m, carry):
        k0 = 2 * m
        issue(k0 + 1, bufs[1])
        scatter(k0, bufs[0])
        issue(k0 + 2, bufs[0])
        scatter(k0 + 1, bufs[1])
        return carry

    lax.fori_loop(0, (NCHUNK - 1) // 2, chunk2, 0, unroll=False)
    scatter(NCHUNK - 1, bufs[0])
    plsc.subcore_barrier()

    for z in range((N_OCH + NS - 1) // NS):
        cidx = sid + NS * z
        @pl.when(cidx < N_OCH)
        def _():
            r0 = cidx * OCH
            pltpu.sync_copy(acc_sh.at[pl.ds(r0, OCH)], zbuf_v)
            pltpu.sync_copy(zbuf_v, out_hbm.at[cid, pl.ds(r0, OCH)])


_sc_segsum = functools.partial(
    pl.kernel,
    out_type=jax.ShapeDtypeStruct((NC, NA, F), jnp.float32),
    mesh=plsc.VectorSubcoreMesh(core_axis_name="c", subcore_axis_name="s"),
    scratch_types=[
        pltpu.VMEM_SHARED((NA, F), jnp.float32),
        pltpu.VMEM((CH, F), jnp.float32),
        pltpu.VMEM((CH,), jnp.int32),
        pltpu.VMEM((CH, F), jnp.float32),
        pltpu.VMEM((CH,), jnp.int32),
        pltpu.VMEM((OCH, F), jnp.float32),
        pltpu.SemaphoreType.DMA,
        pltpu.SemaphoreType.DMA,
    ],
)(_sc_segsum_body)


def _pairout_body(g, pft, wpp, bpp, wp1, wp2, bp, p_out):
    pp = jnp.maximum(
        lax.dot_general(pft[...], wpp[...], _DN_T,
                        preferred_element_type=jnp.float32)
        + bpp[...], 0.0)
    acc = jnp.dot(g[...].astype(jnp.bfloat16), wp1[...],
                  preferred_element_type=jnp.float32)
    acc = acc + jnp.dot(pp.astype(jnp.bfloat16), wp2[...],
                        preferred_element_type=jnp.float32)
    p_out[...] = jnp.maximum(acc + bp[...], 0.0)


def _pairout_stage(g, pair_features_t, wpp, bpp, wp1, wp2, bp):
    blk = 3200
    return pl.pallas_call(
        _pairout_body,
        grid=(NP // blk,),
        in_specs=[
            pl.BlockSpec((blk, F), lambda i: (i, 0)),
            pl.BlockSpec((PF, blk), lambda i: (0, i)),
            pl.BlockSpec((PF, F), lambda i: (0, 0)),
            pl.BlockSpec((1, F), lambda i: (0, 0)),
            pl.BlockSpec((F, F), lambda i: (0, 0)),
            pl.BlockSpec((F, F), lambda i: (0, 0)),
            pl.BlockSpec((1, F), lambda i: (0, 0)),
        ],
        out_specs=pl.BlockSpec((blk, F), lambda i: (i, 0)),
        out_shape=jax.ShapeDtypeStruct((NP, F), jnp.float32),
    )(g, pair_features_t, wpp, bpp, wp1, wp2, bp)


def _atomout_body(aa, parts, wa, ba, a_out):
    pa = parts[0] + parts[1]
    acc = jnp.dot(aa[...], wa[0:F, :], preferred_element_type=jnp.float32)
    acc = acc + jnp.dot(pa, wa[F:2 * F, :], preferred_element_type=jnp.float32)
    a_out[...] = jnp.maximum(acc + ba[...], 0.0)


def _atomout_stage(aa, parts, wa, ba):
    blk = 2000
    return pl.pallas_call(
        _atomout_body,
        grid=(NA // blk,),
        in_specs=[
            pl.BlockSpec((blk, F), lambda i: (i, 0)),
            pl.BlockSpec((NC, blk, F), lambda i: (0, i, 0)),
            pl.BlockSpec((2 * F, F), lambda i: (0, 0)),
            pl.BlockSpec((1, F), lambda i: (0, 0)),
        ],
        out_specs=pl.BlockSpec((blk, F), lambda i: (i, 0)),
        out_shape=jax.ShapeDtypeStruct((NA, F), jnp.float32),
    )(aa, parts, wa, ba)


def kernel(atom_features, pair_features, pair_split, atom_to_pair,
           W_AA, b_AA, W_PA, b_PA, W_A, b_A, W_AP, b_AP, W_PP, b_PP,
           W_P, b_P):
    t1, aa = _atoms_stage(atom_features, W_AP[:F, :], W_AP[F:, :],
                          b_AP.reshape(1, F), W_AA, b_AA.reshape(1, F))
    pft = pair_features.T
    pa = _pa_stage(pft, W_PA, b_PA.reshape(1, F))
    g = _sc_gather(t1, atom_to_pair.T.reshape(-1))
    pa, g = lax.optimization_barrier((pa, g))
    parts = _sc_segsum(pa, pair_split)
    p_out = _pairout_stage(g, pft, W_PP, b_PP.reshape(1, F),
                           W_P[:F, :].astype(jnp.bfloat16),
                           W_P[F:, :].astype(jnp.bfloat16),
                           b_P.reshape(1, F))
    a_out = _atomout_stage(aa, parts, W_A, b_A.reshape(1, F))
    return (a_out, p_out)


# speedup vs baseline: 1.0039x; 1.0039x over previous
import functools

import jax
import jax.numpy as jnp
from jax import lax
from jax.experimental import pallas as pl
from jax.experimental.pallas import tpu as pltpu
from jax.experimental.pallas import tpu_sc as plsc

NA = 10000
NP = 320000
F = 128
PF = 16

NC = 2
NS = 16
NW = NC * NS
PER_TILE = NP // NW
CH = 80
NCHUNK = PER_TILE // CH

OCH = 80
N_OCH = NA // OCH


def _atoms_body(af, w1, w2, bap, waa, baa, t1, aa):
    x = af[...]
    u = jnp.dot(x, w1[...], preferred_element_type=jnp.float32)
    v = jnp.dot(x, w2[...], preferred_element_type=jnp.float32) + bap[...]
    ub = lax.bitcast_convert_type(u.astype(jnp.bfloat16),
                                  jnp.uint16).astype(jnp.uint32)
    vb = lax.bitcast_convert_type(v.astype(jnp.bfloat16),
                                  jnp.uint16).astype(jnp.uint32)
    t1[...] = (ub | (vb << 16)).astype(jnp.int32)
    aa[...] = jnp.maximum(
        jnp.dot(x, waa[...], preferred_element_type=jnp.float32) + baa[...], 0.0)


def _atoms_stage(atom_features, w1, w2, bap, waa, baa):
    blk = 2000
    return pl.pallas_call(
        _atoms_body,
        grid=(NA // blk,),
        in_specs=[
            pl.BlockSpec((blk, F), lambda i: (i, 0)),
            pl.BlockSpec((F, F), lambda i: (0, 0)),
            pl.BlockSpec((F, F), lambda i: (0, 0)),
            pl.BlockSpec((1, F), lambda i: (0, 0)),
            pl.BlockSpec((F, F), lambda i: (0, 0)),
            pl.BlockSpec((1, F), lambda i: (0, 0)),
        ],
        out_specs=[
            pl.BlockSpec((blk, F), lambda i: (i, 0)),
            pl.BlockSpec((blk, F), lambda i: (i, 0)),
        ],
        out_shape=[
            jax.ShapeDtypeStruct((NA, F), jnp.int32),
            jax.ShapeDtypeStruct((NA, F), jnp.float32),
        ],
    )(atom_features, w1, w2, bap, waa, baa)


_DN_T = (((0,), (0,)), ((), ()))


def _pa_body(pft, wpa, bpa, pa):
    pa[...] = jnp.maximum(
        lax.dot_general(pft[...], wpa[...], _DN_T,
                        preferred_element_type=jnp.float32)
        + bpa[...], 0.0)


def _pa_stage(pair_features_t, wpa, bpa):
    blk = 2560
    return pl.pallas_call(
        _pa_body,
        grid=(NP // blk,),
        in_specs=[
            pl.BlockSpec((PF, blk), lambda i: (0, i)),
            pl.BlockSpec((PF, F), lambda i: (0, 0)),
            pl.BlockSpec((1, F), lambda i: (0, 0)),
        ],
        out_specs=pl.BlockSpec((blk, F), lambda i: (i, 0)),
        out_shape=jax.ShapeDtypeStruct((NP, F), jnp.float32),
    )(pair_features_t, wpa, bpa)


def _sc_gather_body(uv1_hbm, ij_hbm, g_hbm,
                    iiall_v, jjall_v,
                    wi0, wj0, out0, wi1, wj1, out1,
                    gsem0, gsem1, osem0, osem1):
    cid = lax.axis_index("c")
    sid = lax.axis_index("s")
    wid = sid * NC + cid
    base = wid * PER_TILE

    pltpu.sync_copy(ij_hbm.at[pl.ds(base, PER_TILE)], iiall_v)
    pltpu.sync_copy(ij_hbm.at[pl.ds(NP + base, PER_TILE)], jjall_v)

    bufs = ((wi0, wj0, out0, gsem0, osem0), (wi1, wj1, out1, gsem1, osem1))

    def issue(k, b):
        wi, wj, _, gsem, _ = b
        c0 = k * CH
        pltpu.async_copy(uv1_hbm.at[iiall_v.at[pl.ds(c0, CH)]], wi, gsem)
        pltpu.async_copy(uv1_hbm.at[jjall_v.at[pl.ds(c0, CH)]], wj, gsem)

    def compute(k, b):
        wi, wj, out, gsem, osem = b
        c0 = k * CH
        off = base + c0
        pltpu.make_async_copy(uv1_hbm.at[iiall_v.at[pl.ds(c0, CH)]], wi,
                              gsem).wait()
        pltpu.make_async_copy(uv1_hbm.at[jjall_v.at[pl.ds(c0, CH)]], wj,
                              gsem).wait()

        @pl.when(k >= 2)
        def _():
            pltpu.make_async_copy(out, g_hbm.at[pl.ds(off, CH)], osem).wait()

        @plsc.parallel_loop(0, CH, 1, unroll=2)
        def pair(p):
            for h in range(F // 16):
                s = h * 16
                a = wi[p, pl.ds(s, 16)]
                bv = wj[p, pl.ds(s, 16)]
                ui = lax.bitcast_convert_type(a << 16, jnp.float32)
                vpi = lax.bitcast_convert_type(a & jnp.int32(-65536),
                                               jnp.float32)
                uj = lax.bitcast_convert_type(bv << 16, jnp.float32)
                vpj = lax.bitcast_convert_type(bv & jnp.int32(-65536),
                                               jnp.float32)
                out[p, pl.ds(s, 16)] = (jnp.maximum(ui + vpj, 0.0)
                                        + jnp.maximum(uj + vpi, 0.0))
        pltpu.async_copy(out, g_hbm.at[pl.ds(off, CH)], osem)

    issue(0, bufs[0])

    def body(m, carry):
        k0 = 2 * m
        issue(k0 + 1, bufs[1])
        compute(k0, bufs[0])
        issue(k0 + 2, bufs[0])
        compute(k0 + 1, bufs[1])
        return carry

    lax.fori_loop(0, (NCHUNK - 1) // 2, body, 0, unroll=False)
    compute(NCHUNK - 1, bufs[0])

    pltpu.make_async_copy(out0, g_hbm.at[pl.ds(base, CH)], osem0).wait()
    pltpu.make_async_copy(out1, g_hbm.at[pl.ds(base, CH)], osem1).wait()


_sc_gather = functools.partial(
    pl.kernel,
    out_type=jax.ShapeDtypeStruct((NP, F), jnp.float32),
    mesh=plsc.VectorSubcoreMesh(core_axis_name="c", subcore_axis_name="s"),
    scratch_types=[
        pltpu.VMEM((PER_TILE,), jnp.int32),
        pltpu.VMEM((PER_TILE,), jnp.int32),
        pltpu.VMEM((CH, F), jnp.int32),
        pltpu.VMEM((CH, F), jnp.int32),
        pltpu.VMEM((CH, F), jnp.float32),
        pltpu.VMEM((CH, F), jnp.int32),
        pltpu.VMEM((CH, F), jnp.int32),
        pltpu.VMEM((CH, F), jnp.float32),
        pltpu.SemaphoreType.DMA,
        pltpu.SemaphoreType.DMA,
        pltpu.SemaphoreType.DMA,
        pltpu.SemaphoreType.DMA,
    ],
)(_sc_gather_body)


def _sc_segsum_body(pa_hbm, split_hbm, out_hbm, acc_sh,
                    rows0_v, seg0_v, rows1_v, seg1_v, zbuf_v, lsem0, lsem1):
    cid = lax.axis_index("c")
    sid = lax.axis_index("s")
    wid = sid * NC + cid

    def zrow(p, c):
        for h in range(F // 16):
            zbuf_v[p, pl.ds(h * 16, 16)] = jnp.zeros((16,), jnp.float32)
        return c

    lax.fori_loop(0, OCH, zrow, 0, unroll=False)
    for z in range((N_OCH + NS - 1) // NS):
        cidx = sid + NS * z
        @pl.when(cidx < N_OCH)
        def _():
            pltpu.sync_copy(zbuf_v, acc_sh.at[pl.ds(cidx * OCH, OCH)])
    plsc.subcore_barrier()

    base = wid * PER_TILE
    bufs = ((rows0_v, seg0_v, lsem0), (rows1_v, seg1_v, lsem1))

    def issue(k, b):
        rows, seg, lsem = b
        off = base + k * CH
        pltpu.async_copy(pa_hbm.at[pl.ds(off, CH)], rows, lsem)
        pltpu.async_copy(split_hbm.at[pl.ds(off, CH)], seg, lsem)

    def scatter(k, b):
        rows, seg, lsem = b
        off = base + k * CH
        pltpu.make_async_copy(pa_hbm.at[pl.ds(off, CH)], rows, lsem).wait()
        pltpu.make_async_copy(split_hbm.at[pl.ds(off, CH)], seg, lsem).wait()
        pltpu.sync_copy(rows, acc_sh.at[seg], add=True)

    issue(0, bufs[0])

    def chunk2(m, carry):
        k0 = 2 * m
        issue(k0 + 1, bufs[1])
        scatter(k0, bufs[0])
        issue(k0 + 2, bufs[0])
        scatter(k0 + 1, bufs[1])
        return carry

    lax.fori_loop(0, (NCHUNK - 1) // 2, chunk2, 0, unroll=False)
    scatter(NCHUNK - 1, bufs[0])
    plsc.subcore_barrier()

    for z in range((N_OCH + NS - 1) // NS):
        cidx = sid + NS * z
        @pl.when(cidx < N_OCH)
        def _():
            r0 = cidx * OCH
            pltpu.sync_copy(acc_sh.at[pl.ds(r0, OCH)], zbuf_v)
            pltpu.sync_copy(zbuf_v, out_hbm.at[cid, pl.ds(r0, OCH)])


_sc_segsum = functools.partial(
    pl.kernel,
    out_type=jax.ShapeDtypeStruct((NC, NA, F), jnp.float32),
    mesh=plsc.VectorSubcoreMesh(core_axis_name="c", subcore_axis_name="s"),
    scratch_types=[
        pltpu.VMEM_SHARED((NA, F), jnp.float32),
        pltpu.VMEM((CH, F), jnp.float32),
        pltpu.VMEM((CH,), jnp.int32),
        pltpu.VMEM((CH, F), jnp.float32),
        pltpu.VMEM((CH,), jnp.int32),
        pltpu.VMEM((OCH, F), jnp.float32),
        pltpu.SemaphoreType.DMA,
        pltpu.SemaphoreType.DMA,
    ],
)(_sc_segsum_body)


def _pairout_body(g, pft, wpp, bpp, wp1, wp2, bp, p_out):
    pp = jnp.maximum(
        lax.dot_general(pft[...], wpp[...], _DN_T,
                        preferred_element_type=jnp.float32)
        + bpp[...], 0.0)
    acc = jnp.dot(g[...].astype(jnp.bfloat16), wp1[...],
                  preferred_element_type=jnp.float32)
    acc = acc + jnp.dot(pp.astype(jnp.bfloat16), wp2[...],
                        preferred_element_type=jnp.float32)
    p_out[...] = jnp.maximum(acc + bp[...], 0.0)


def _pairout_stage(g, pair_features_t, wpp, bpp, wp1, wp2, bp):
    blk = 3200
    return pl.pallas_call(
        _pairout_body,
        grid=(NP // blk,),
        in_specs=[
            pl.BlockSpec((blk, F), lambda i: (i, 0)),
            pl.BlockSpec((PF, blk), lambda i: (0, i)),
            pl.BlockSpec((PF, F), lambda i: (0, 0)),
            pl.BlockSpec((1, F), lambda i: (0, 0)),
            pl.BlockSpec((F, F), lambda i: (0, 0)),
            pl.BlockSpec((F, F), lambda i: (0, 0)),
            pl.BlockSpec((1, F), lambda i: (0, 0)),
        ],
        out_specs=pl.BlockSpec((blk, F), lambda i: (i, 0)),
        out_shape=jax.ShapeDtypeStruct((NP, F), jnp.float32),
    )(g, pair_features_t, wpp, bpp, wp1, wp2, bp)


def _atomout_body(aa, parts, wa, ba, a_out):
    pa = parts[0] + parts[1]
    acc = jnp.dot(aa[...], wa[0:F, :], preferred_element_type=jnp.float32)
    acc = acc + jnp.dot(pa, wa[F:2 * F, :], preferred_element_type=jnp.float32)
    a_out[...] = jnp.maximum(acc + ba[...], 0.0)


def _atomout_stage(aa, parts, wa, ba):
    blk = 2000
    return pl.pallas_call(
        _atomout_body,
        grid=(NA // blk,),
        in_specs=[
            pl.BlockSpec((blk, F), lambda i: (i, 0)),
            pl.BlockSpec((NC, blk, F), lambda i: (0, i, 0)),
            pl.BlockSpec((2 * F, F), lambda i: (0, 0)),
            pl.BlockSpec((1, F), lambda i: (0, 0)),
        ],
        out_specs=pl.BlockSpec((blk, F), lambda i: (i, 0)),
        out_shape=jax.ShapeDtypeStruct((NA, F), jnp.float32),
    )(aa, parts, wa, ba)


def kernel(atom_features, pair_features, pair_split, atom_to_pair,
           W_AA, b_AA, W_PA, b_PA, W_A, b_A, W_AP, b_AP, W_PP, b_PP,
           W_P, b_P):
    t1, aa = _atoms_stage(atom_features, W_AP[:F, :], W_AP[F:, :],
                          b_AP.reshape(1, F), W_AA, b_AA.reshape(1, F))
    pft = pair_features.T
    pa = _pa_stage(pft, W_PA, b_PA.reshape(1, F))
    g = _sc_gather(t1, atom_to_pair.T.reshape(-1))
    pa, g = lax.optimization_barrier((pa, g))
    parts = _sc_segsum(pa, pair_split)
    p_out = _pairout_stage(g, pft, W_PP, b_PP.reshape(1, F),
                           W_P[:F, :].astype(jnp.bfloat16),
                           W_P[F:, :].astype(jnp.bfloat16),
                           b_P.reshape(1, F))
    a_out = _atomout_stage(aa, parts, W_A, b_A.reshape(1, F))
    return (a_out, p_out)
